# baseline (speedup 1.0000x reference)
import functools

import jax
import jax.numpy as jnp
from jax import lax
from jax.experimental import pallas as pl
from jax.experimental.pallas import tpu as pltpu
from jax.experimental.pallas import tpu_sc as plsc

B, S, D = 4, 2048, 1024
NC, NS, L = 2, 16, 16
NW = NC * NS
ROWS_PW = S // NW
CHUNK = 32
NCH = ROWS_PW // CHUNK
NXB = 2
NPB = 1
T = NCH * B

_mesh = plsc.VectorSubcoreMesh(
    core_axis_name="c", subcore_axis_name="s", num_cores=NC, num_subcores=NS
)


@functools.partial(
    pl.kernel,
    out_type=jax.ShapeDtypeStruct((B, S, D), jnp.float32),
    mesh=_mesh,
    scratch_types=[
        pltpu.VMEM((NXB, CHUNK, D), jnp.float32),
        pltpu.VMEM((NPB, CHUNK, D), jnp.float32),
        pltpu.SemaphoreType.DMA((NXB,)),
        pltpu.SemaphoreType.DMA((NXB,)),
        pltpu.SemaphoreType.DMA((NPB,)),
    ],
)
def _pe_add(x_hbm, pe_hbm, out_hbm, xbuf, pbuf, sin, sout, spe):
    wid = lax.axis_index("s") * NC + lax.axis_index("c")
    base = wid * ROWS_PW
    vecs_per_row = D // L

    def start_in(t):
        c, b = t // B, t % B
        pltpu.async_copy(
            x_hbm.at[b, pl.ds(base + c * CHUNK, CHUNK), :],
            xbuf.at[t % NXB], sin.at[t % NXB])

    def wait_in(t):
        c, b = t // B, t % B
        pltpu.make_async_copy(
            x_hbm.at[b, pl.ds(base + c * CHUNK, CHUNK), :],
            xbuf.at[t % NXB], sin.at[t % NXB]).wait()

    def start_out(t):
        c, b = t // B, t % B
        pltpu.async_copy(
            xbuf.at[t % NXB],
            out_hbm.at[b, pl.ds(base + c * CHUNK, CHUNK), :],
            sout.at[t % NXB])

    def wait_out(t):
        c, b = t // B, t % B
        pltpu.make_async_copy(
            xbuf.at[t % NXB],
            out_hbm.at[b, pl.ds(base + c * CHUNK, CHUNK), :],
            sout.at[t % NXB]).wait()

    def start_pe(c):
        pltpu.async_copy(
            pe_hbm.at[pl.ds(base + c * CHUNK, CHUNK), :],
            pbuf.at[c % NPB], spe.at[c % NPB])

    def wait_pe(c):
        pltpu.make_async_copy(
            pe_hbm.at[pl.ds(base + c * CHUNK, CHUNK), :],
            pbuf.at[c % NPB], spe.at[c % NPB]).wait()

    start_pe(0)
    for t in range(NXB - 1):
        start_in(t)

    def tile_step(t, carry):
        c, b = t // B, t % B

        if NPB > 1:
            @pl.when(b == 0)
            def _():
                wait_pe(c)

                @pl.when(c + 1 < NCH)
                def _():
                    start_pe(c + 1)
        else:
            @pl.when(b == 0)
            def _():
                @pl.when(c >= 1)
                def _():
                    start_pe(c)

                wait_pe(c)

        wait_in(t)

        @plsc.parallel_loop(0, CHUNK * vecs_per_row, 1, unroll=8)
        def _(k):
            r = k // vecs_per_row
            col = (k % vecs_per_row) * L
            plsc.addupdate(
                xbuf.at[t % NXB, r, pl.ds(col, L)],
                pbuf[c % NPB, r, pl.ds(col, L)])

        start_out(t)

        @pl.when(t + NXB - 1 < T)
        def _():
            @pl.when(t >= 1)
            def _():
                wait_out(t - 1)

            start_in(t + NXB - 1)

        return carry

    lax.fori_loop(0, T, tile_step, 0)

    def drain(t, carry):
        wait_out(t)
        return carry

    lax.fori_loop(T - NXB, T, drain, 0)


def kernel(x, embedding_weight):
    return _pe_add(x, embedding_weight)


# speedup vs baseline: 1.3676x; 1.3676x over previous
import functools

import jax
import jax.numpy as jnp
from jax import lax
from jax.experimental import pallas as pl
from jax.experimental.pallas import tpu as pltpu
from jax.experimental.pallas import tpu_sc as plsc

B, S, D = 4, 2048, 1024
NC, NS, L = 2, 16, 16
NW = NC * NS
ROWS_PW = S // NW
CHUNK = 16
NCH = ROWS_PW // CHUNK
NXB = 5
NPB = 2
T = NCH * B

_mesh = plsc.VectorSubcoreMesh(
    core_axis_name="c", subcore_axis_name="s", num_cores=NC, num_subcores=NS
)


@functools.partial(
    pl.kernel,
    out_type=jax.ShapeDtypeStruct((B, S, D), jnp.float32),
    mesh=_mesh,
    scratch_types=[
        pltpu.VMEM((NXB, CHUNK, D), jnp.float32),
        pltpu.VMEM((NPB, CHUNK, D), jnp.float32),
        pltpu.SemaphoreType.DMA((NXB,)),
        pltpu.SemaphoreType.DMA((NXB,)),
        pltpu.SemaphoreType.DMA((NPB,)),
    ],
)
def _pe_add(x_hbm, pe_hbm, out_hbm, xbuf, pbuf, sin, sout, spe):
    wid = lax.axis_index("s") * NC + lax.axis_index("c")
    base = wid * ROWS_PW
    vecs_per_row = D // L

    def start_in(t):
        c, b = t // B, t % B
        pltpu.async_copy(
            x_hbm.at[b, pl.ds(base + c * CHUNK, CHUNK), :],
            xbuf.at[t % NXB], sin.at[t % NXB])

    def wait_in(t):
        c, b = t // B, t % B
        pltpu.make_async_copy(
            x_hbm.at[b, pl.ds(base + c * CHUNK, CHUNK), :],
            xbuf.at[t % NXB], sin.at[t % NXB]).wait()

    def start_out(t):
        c, b = t // B, t % B
        pltpu.async_copy(
            xbuf.at[t % NXB],
            out_hbm.at[b, pl.ds(base + c * CHUNK, CHUNK), :],
            sout.at[t % NXB])

    def wait_out(t):
        c, b = t // B, t % B
        pltpu.make_async_copy(
            xbuf.at[t % NXB],
            out_hbm.at[b, pl.ds(base + c * CHUNK, CHUNK), :],
            sout.at[t % NXB]).wait()

    def start_pe(c):
        pltpu.async_copy(
            pe_hbm.at[pl.ds(base + c * CHUNK, CHUNK), :],
            pbuf.at[c % NPB], spe.at[c % NPB])

    def wait_pe(c):
        pltpu.make_async_copy(
            pe_hbm.at[pl.ds(base + c * CHUNK, CHUNK), :],
            pbuf.at[c % NPB], spe.at[c % NPB]).wait()

    start_pe(0)
    for t in range(NXB - 1):
        start_in(t)

    def tile_step(t, carry):
        c, b = t // B, t % B

        if NPB > 1:
            @pl.when(b == 0)
            def _():
                wait_pe(c)

                @pl.when(c + 1 < NCH)
                def _():
                    start_pe(c + 1)
        else:
            @pl.when(b == 0)
            def _():
                @pl.when(c >= 1)
                def _():
                    start_pe(c)

                wait_pe(c)

        wait_in(t)

        @plsc.parallel_loop(0, CHUNK * vecs_per_row, 1, unroll=8)
        def _(k):
            r = k // vecs_per_row
            col = (k % vecs_per_row) * L
            plsc.addupdate(
                xbuf.at[t % NXB, r, pl.ds(col, L)],
                pbuf[c % NPB, r, pl.ds(col, L)])

        start_out(t)

        @pl.when(t + NXB - 1 < T)
        def _():
            @pl.when(t >= 1)
            def _():
                wait_out(t - 1)

            start_in(t + NXB - 1)

        return carry

    lax.fori_loop(0, T, tile_step, 0)

    def drain(t, carry):
        wait_out(t)
        return carry

    lax.fori_loop(T - NXB, T, drain, 0)


def kernel(x, embedding_weight):
    return _pe_add(x, embedding_weight)


# speedup vs baseline: 1.3843x; 1.0122x over previous
import functools

import jax
import jax.numpy as jnp
from jax import lax
from jax.experimental import pallas as pl
from jax.experimental.pallas import tpu as pltpu
from jax.experimental.pallas import tpu_sc as plsc

B, S, D = 4, 2048, 1024
NC, NS, L = 2, 16, 16
NW = NC * NS
ROWS_PW = S // NW
CHUNK = 8
NCH = ROWS_PW // CHUNK
NXB = 10
NPB = 3
T = NCH * B

_mesh = plsc.VectorSubcoreMesh(
    core_axis_name="c", subcore_axis_name="s", num_cores=NC, num_subcores=NS
)


@functools.partial(
    pl.kernel,
    out_type=jax.ShapeDtypeStruct((B, S, D), jnp.float32),
    mesh=_mesh,
    scratch_types=[
        pltpu.VMEM((NXB, CHUNK, D), jnp.float32),
        pltpu.VMEM((NPB, CHUNK, D), jnp.float32),
        pltpu.SemaphoreType.DMA((NXB,)),
        pltpu.SemaphoreType.DMA((NXB,)),
        pltpu.SemaphoreType.DMA((NPB,)),
    ],
)
def _pe_add(x_hbm, pe_hbm, out_hbm, xbuf, pbuf, sin, sout, spe):
    wid = lax.axis_index("s") * NC + lax.axis_index("c")
    base = wid * ROWS_PW
    vecs_per_row = D // L

    def start_in(t):
        c, b = t // B, t % B
        pltpu.async_copy(
            x_hbm.at[b, pl.ds(base + c * CHUNK, CHUNK), :],
            xbuf.at[t % NXB], sin.at[t % NXB])

    def wait_in(t):
        c, b = t // B, t % B
        pltpu.make_async_copy(
            x_hbm.at[b, pl.ds(base + c * CHUNK, CHUNK), :],
            xbuf.at[t % NXB], sin.at[t % NXB]).wait()

    def start_out(t):
        c, b = t // B, t % B
        pltpu.async_copy(
            xbuf.at[t % NXB],
            out_hbm.at[b, pl.ds(base + c * CHUNK, CHUNK), :],
            sout.at[t % NXB])

    def wait_out(t):
        c, b = t // B, t % B
        pltpu.make_async_copy(
            xbuf.at[t % NXB],
            out_hbm.at[b, pl.ds(base + c * CHUNK, CHUNK), :],
            sout.at[t % NXB]).wait()

    def start_pe(c):
        pltpu.async_copy(
            pe_hbm.at[pl.ds(base + c * CHUNK, CHUNK), :],
            pbuf.at[c % NPB], spe.at[c % NPB])

    def wait_pe(c):
        pltpu.make_async_copy(
            pe_hbm.at[pl.ds(base + c * CHUNK, CHUNK), :],
            pbuf.at[c % NPB], spe.at[c % NPB]).wait()

    start_pe(0)
    for t in range(NXB - 1):
        start_in(t)

    def tile_step(t, carry):
        c, b = t // B, t % B

        if NPB > 1:
            @pl.when(b == 0)
            def _():
                wait_pe(c)

                @pl.when(c + 1 < NCH)
                def _():
                    start_pe(c + 1)
        else:
            @pl.when(b == 0)
            def _():
                @pl.when(c >= 1)
                def _():
                    start_pe(c)

                wait_pe(c)

        wait_in(t)

        @plsc.parallel_loop(0, CHUNK * vecs_per_row, 1, unroll=8)
        def _(k):
            r = k // vecs_per_row
            col = (k % vecs_per_row) * L
            plsc.addupdate(
                xbuf.at[t % NXB, r, pl.ds(col, L)],
                pbuf[c % NPB, r, pl.ds(col, L)])

        start_out(t)

        @pl.when(t + NXB - 1 < T)
        def _():
            @pl.when(t >= 1)
            def _():
                wait_out(t - 1)

            start_in(t + NXB - 1)

        return carry

    lax.fori_loop(0, T, tile_step, 0)

    def drain(t, carry):
        wait_out(t)
        return carry

    lax.fori_loop(T - NXB, T, drain, 0)


def kernel(x, embedding_weight):
    return _pe_add(x, embedding_weight)
